# baseline (speedup 1.0000x reference)
import jax
import jax.numpy as jnp
from jax import lax
from jax.experimental import pallas as pl
from jax.experimental.pallas import tpu as pltpu
from jax.experimental.pallas import tpu_sc as plsc
from jax._src.pallas import mpmd
from jax._src.pallas import core as _pallas_core
from jax._src.pallas.mosaic import core as _tpu_core

_T = 8192
_D = 768

_TEC_ROWS = 4096
_NW = 32
_V_RPW = _TEC_ROWS // _NW
_V_CH = 16
_V_NCH = _V_RPW // _V_CH
_V_NBUF = 8
_V_LAG = 4

_SCS_ROWS = _T - _TEC_ROWS
_NC = 2
_S_RPC = _SCS_ROWS // _NC
_S_CH = 256
_S_NCH = _S_RPC // _S_CH
_S_NBUF = 2
_S_LAG = 1


def _ring_copy(src_hbm, dst_hbm, buf, rsems, wsems, base, ch, nch, nbuf, lag):
    def rd(i, slot):
        return pltpu.make_async_copy(
            src_hbm.at[pl.ds(base + i * ch, ch)], buf.at[slot], rsems.at[slot])

    def wr(i, slot):
        return pltpu.make_async_copy(
            buf.at[slot], dst_hbm.at[pl.ds(base + i * ch, ch)], wsems.at[slot])

    for j in range(min(nbuf - lag, nch)):
        rd(j, j).start()

    def step(i, carry):
        @pl.when(i >= lag)
        def _():
            wr(i - lag, lax.rem(i - lag, nbuf)).wait()

        nxt = i + nbuf - lag

        @pl.when(nxt < nch)
        def _():
            rd(nxt, lax.rem(nxt, nbuf)).start()

        slot = lax.rem(i, nbuf)
        rd(i, slot).wait()
        wr(i, slot).start()
        return carry

    lax.fori_loop(0, nch, step, 0, unroll=False)

    def drain(i, carry):
        wr(i, lax.rem(i, nbuf)).wait()
        return carry

    lax.fori_loop(max(0, nch - lag), nch, drain, 0, unroll=False)


def _make_sc_copy():
    scalar_mesh = plsc.ScalarSubcoreMesh(axis_name="c", num_cores=_NC)
    vector_mesh = plsc.VectorSubcoreMesh(
        core_axis_name="c", subcore_axis_name="s")

    def scs_fn(wpe_hbm, out_hbm, vbuf, vrsems, vwsems, sbuf, srsems, swsems):
        del vbuf, vrsems, vwsems
        cid = lax.axis_index("c")
        base = _TEC_ROWS + cid * _S_RPC
        _ring_copy(wpe_hbm, out_hbm, sbuf, srsems, swsems,
                   base, _S_CH, _S_NCH, _S_NBUF, _S_LAG)

    def tec_fn(wpe_hbm, out_hbm, vbuf, vrsems, vwsems, sbuf, srsems, swsems):
        del sbuf, srsems, swsems
        wid = lax.axis_index("s") * 2 + lax.axis_index("c")
        base = wid * _V_RPW
        _ring_copy(wpe_hbm, out_hbm, vbuf, vrsems, vwsems,
                   base, _V_CH, _V_NCH, _V_NBUF, _V_LAG)

    return mpmd.mpmd_map(
        [(scalar_mesh, scs_fn), (vector_mesh, tec_fn)],
        out_types=jax.ShapeDtypeStruct((_T, _D), jnp.float32),
        scratch_types=[
            _pallas_core.CoreMemorySpace(_tpu_core.MemorySpace.VMEM, vector_mesh)(
                (_V_NBUF, _V_CH, _D), jnp.float32),
            _pallas_core.CoreMemorySpace(
                _tpu_core.MemorySpace.SEMAPHORE, vector_mesh)(
                (_V_NBUF,), _tpu_core.SemaphoreType.DMA.dtype),
            _pallas_core.CoreMemorySpace(
                _tpu_core.MemorySpace.SEMAPHORE, vector_mesh)(
                (_V_NBUF,), _tpu_core.SemaphoreType.DMA.dtype),
            pltpu.MemorySpace.VMEM_SHARED((_S_NBUF, _S_CH, _D), jnp.float32),
            _pallas_core.CoreMemorySpace(
                _tpu_core.MemorySpace.SEMAPHORE, scalar_mesh)(
                (_S_NBUF,), _tpu_core.SemaphoreType.DMA.dtype),
            _pallas_core.CoreMemorySpace(
                _tpu_core.MemorySpace.SEMAPHORE, scalar_mesh)(
                (_S_NBUF,), _tpu_core.SemaphoreType.DMA.dtype),
        ],
    )


_sc_copy = _make_sc_copy()


def kernel(idx, wpe):
    del idx
    return _sc_copy(wpe)
